# baseline (speedup 1.0000x reference)
import jax
import jax.numpy as jnp
from jax import lax
from jax.experimental import pallas as pl
from jax.experimental.pallas import tpu as pltpu
from jax.experimental.pallas import tpu_sc as plsc

AUTHOR_DIM = 1000000
AUTHOR_PAD = 1000064
EMBED_DIM = 32
NUM_WORKERS = 32
NB_FULL = 7812
TAIL_BASE = NB_FULL * 128

B_DIM = 16384
S_DIM = 50
B_PER_W = 512
IDX_PER_W = B_PER_W * S_DIM


def _iota16():
    return lax.iota(jnp.int32, 16)


def _splat(v):
    return jnp.full((16,), v, jnp.int32)


def _transpose_block(src, dst, n_rows):
    for r in range(n_rows):
        for g in range(8):
            rows = _iota16() + 16 * (g & 1)
            cols = _splat(r * 4 + g // 2)
            vals = plsc.load_gather(src, [rows, cols])
            dst[r, pl.ds(g * 16, 16)] = vals


def _body_a(tt_hbm, tl_hbm, ibuf, obuf, tbuf_i, tbuf_o):
    wid = lax.axis_index("s") * 2 + lax.axis_index("c")

    def step(k, carry):
        c = wid + 32 * k
        off = pl.multiple_of(c * 128, 128)
        pltpu.sync_copy(tt_hbm.at[:, pl.ds(off, 128)], ibuf)
        _transpose_block(ibuf, obuf, 32)
        row0 = pl.multiple_of(c * 32, 32)
        pltpu.sync_copy(obuf, tl_hbm.at[pl.ds(row0, 32)])
        return carry

    lax.fori_loop(0, 244, step, 0)

    @pl.when(wid < 4)
    def _():
        step(244, 0)

    @pl.when(wid == 4)
    def _():
        pltpu.sync_copy(tt_hbm.at[:, pl.ds(TAIL_BASE, 64)], tbuf_i)
        _transpose_block(tbuf_i, tbuf_o, 16)
        pltpu.sync_copy(tbuf_o, tl_hbm.at[pl.ds(NB_FULL * 32, 16)])


NBUF = 4


def _body_b(idx_hbm, t2_hbm, out_hbm, idx_v, *rest):
    idxcols = rest[:NBUF]
    rows = rest[NBUF:2 * NBUF]
    obufs = rest[2 * NBUF:3 * NBUF]
    gsems = rest[3 * NBUF:4 * NBUF]

    wid = lax.axis_index("s") * 2 + lax.axis_index("c")
    pltpu.sync_copy(idx_hbm.at[pl.ds(wid * IDX_PER_W, IDX_PER_W)], idx_v)

    def build_and_fire(L, b):
        bb = L // S_DIM
        s = L % S_DIM
        for h in range(8):
            pos = (bb * 128 + 16 * h + _iota16()) * S_DIM + s
            idxcols[b][pl.ds(16 * h, 16)] = plsc.load_gather(idx_v, [pos])
        return pltpu.async_copy(t2_hbm.at[idxcols[b]], rows[b], gsems[b])

    def drain_and_write(L, b):
        pltpu.make_async_copy(t2_hbm.at[idxcols[b]], rows[b], gsems[b]).wait()
        bb = L // S_DIM
        s = L % S_DIM
        for eg in range(4):
            for es in range(8):
                for h in range(8):
                    vals = plsc.load_gather(
                        rows[b], [_iota16() + 16 * h, _splat(eg * 8 + es)]
                    )
                    obufs[b][eg, es, pl.ds(16 * h, 16)] = vals
            pltpu.sync_copy(obufs[b].at[eg], out_hbm.at[s, eg, wid * 4 + bb])

    for b in range(NBUF):
        build_and_fire(b, b)

    def step(t, carry):
        for b in range(NBUF):
            L = NBUF * t + b
            drain_and_write(L, b)
            build_and_fire(L + NBUF, b)
        return carry

    n_iter = 4 * S_DIM // NBUF - 1
    lax.fori_loop(0, n_iter, step, 0)
    for b in range(NBUF):
        drain_and_write(NBUF * n_iter + b, b)


@jax.jit
def kernel(inputs, table):
    mesh = plsc.VectorSubcoreMesh(core_axis_name="c", subcore_axis_name="s")

    t_lin = pl.kernel(
        _body_a,
        out_type=jax.ShapeDtypeStruct((AUTHOR_PAD // 4, 128), jnp.float32),
        mesh=mesh,
        scratch_types=[
            pltpu.VMEM((32, 128), jnp.float32),
            pltpu.VMEM((32, 128), jnp.float32),
            pltpu.VMEM((32, 64), jnp.float32),
            pltpu.VMEM((16, 128), jnp.float32),
        ],
        compiler_params=pltpu.CompilerParams(
            use_tc_tiling_on_sc=True, needs_layout_passes=False
        ),
    )(table.T)
    t2 = t_lin.reshape(AUTHOR_PAD, EMBED_DIM)

    idx_flat = inputs.reshape(-1)
    out5 = pl.kernel(
        _body_b,
        out_type=jax.ShapeDtypeStruct((S_DIM, 4, 128, 8, 128), jnp.float32),
        mesh=mesh,
        scratch_types=(
            [pltpu.VMEM((IDX_PER_W,), jnp.int32)]
            + [pltpu.VMEM((128,), jnp.int32)] * NBUF
            + [pltpu.VMEM((128, EMBED_DIM), jnp.float32)] * NBUF
            + [pltpu.VMEM((4, 8, 128), jnp.float32)] * NBUF
            + [pltpu.SemaphoreType.DMA] * NBUF
        ),
        compiler_params=pltpu.CompilerParams(
            use_tc_tiling_on_sc=False, needs_layout_passes=False
        ),
    )(idx_flat, t2)
    return out5.transpose(2, 4, 0, 1, 3).reshape(B_DIM, S_DIM, EMBED_DIM)


# speedup vs baseline: 1.1758x; 1.1758x over previous
import jax
import jax.numpy as jnp
from jax import lax
from jax.experimental import pallas as pl
from jax.experimental.pallas import tpu as pltpu
from jax.experimental.pallas import tpu_sc as plsc

AUTHOR_DIM = 1000000
AUTHOR_PAD = 1000064
EMBED_DIM = 32
NUM_WORKERS = 32
NB_FULL = 7812
TAIL_BASE = NB_FULL * 128

B_DIM = 16384
S_DIM = 50
B_PER_W = 512
IDX_PER_W = B_PER_W * S_DIM


def _iota16():
    return lax.iota(jnp.int32, 16)


def _splat(v):
    return jnp.full((16,), v, jnp.int32)


def _transpose_block(src, dst, n_rows):
    for r in range(n_rows):
        for g in range(8):
            rows = _iota16() + 16 * (g & 1)
            cols = _splat(r * 4 + g // 2)
            vals = plsc.load_gather(src, [rows, cols])
            dst[r, pl.ds(g * 16, 16)] = vals


def _body_a(tt_hbm, tl_hbm, ibuf, obuf, tbuf_i, tbuf_o):
    wid = lax.axis_index("s") * 2 + lax.axis_index("c")

    def step(k, carry):
        c = wid + 32 * k
        off = pl.multiple_of(c * 128, 128)
        pltpu.sync_copy(tt_hbm.at[:, pl.ds(off, 128)], ibuf)
        _transpose_block(ibuf, obuf, 32)
        row0 = pl.multiple_of(c * 32, 32)
        pltpu.sync_copy(obuf, tl_hbm.at[pl.ds(row0, 32)])
        return carry

    lax.fori_loop(0, 244, step, 0)

    @pl.when(wid < 4)
    def _():
        step(244, 0)

    @pl.when(wid == 4)
    def _():
        pltpu.sync_copy(tt_hbm.at[:, pl.ds(TAIL_BASE, 64)], tbuf_i)
        _transpose_block(tbuf_i, tbuf_o, 16)
        pltpu.sync_copy(tbuf_o, tl_hbm.at[pl.ds(NB_FULL * 32, 16)])


NBUF = 4


def _body_b(idx_hbm, t2_hbm, out_hbm, idx_v, *rest):
    idxcols = rest[:NBUF]
    rows = rest[NBUF:2 * NBUF]
    obufs = rest[2 * NBUF:3 * NBUF]
    gsems = rest[3 * NBUF:4 * NBUF]

    wid = lax.axis_index("s") * 2 + lax.axis_index("c")
    pltpu.sync_copy(idx_hbm.at[pl.ds(wid * IDX_PER_W, IDX_PER_W)], idx_v)

    def build_and_fire(L, b):
        bb = L // S_DIM
        s = L % S_DIM
        for h in range(8):
            pos = (bb * 128 + 16 * h + _iota16()) * S_DIM + s
            idxcols[b][pl.ds(16 * h, 16)] = plsc.load_gather(idx_v, [pos])
        return pltpu.async_copy(t2_hbm.at[idxcols[b]], rows[b], gsems[b])

    def drain_and_write(L, b):
        pltpu.make_async_copy(t2_hbm.at[idxcols[b]], rows[b], gsems[b]).wait()
        bb = L // S_DIM
        s = L % S_DIM
        for eg in range(4):
            for es in range(8):
                for h in range(8):
                    vals = plsc.load_gather(
                        rows[b], [_iota16() + 16 * h, _splat(eg * 8 + es)]
                    )
                    obufs[b][eg, es, pl.ds(16 * h, 16)] = vals
            pltpu.sync_copy(obufs[b].at[eg], out_hbm.at[s, eg, wid * 4 + bb])

    for b in range(NBUF):
        build_and_fire(b, b)

    def step(t, carry):
        for b in range(NBUF):
            L = NBUF * t + b
            drain_and_write(L, b)
            build_and_fire(L + NBUF, b)
        return carry

    n_iter = 4 * S_DIM // NBUF - 1
    lax.fori_loop(0, n_iter, step, 0)
    for b in range(NBUF):
        drain_and_write(NBUF * n_iter + b, b)


@jax.jit
def kernel(inputs, table):
    mesh = plsc.VectorSubcoreMesh(core_axis_name="c", subcore_axis_name="s")

    t_lin = pl.kernel(
        _body_a,
        out_type=jax.ShapeDtypeStruct((AUTHOR_PAD // 4, 128), jnp.float32),
        mesh=mesh,
        scratch_types=[
            pltpu.VMEM((32, 128), jnp.float32),
            pltpu.VMEM((32, 128), jnp.float32),
            pltpu.VMEM((32, 64), jnp.float32),
            pltpu.VMEM((16, 128), jnp.float32),
        ],
        compiler_params=pltpu.CompilerParams(
            use_tc_tiling_on_sc=True, needs_layout_passes=False
        ),
    )(table.T)
    t2 = t_lin.reshape(AUTHOR_PAD, EMBED_DIM)
    t2 = jnp.pad(table, ((0, AUTHOR_PAD - AUTHOR_DIM), (0, 0)))

    idx_flat = inputs.reshape(-1)
    out5 = pl.kernel(
        _body_b,
        out_type=jax.ShapeDtypeStruct((S_DIM, 4, 128, 8, 128), jnp.float32),
        mesh=mesh,
        scratch_types=(
            [pltpu.VMEM((IDX_PER_W,), jnp.int32)]
            + [pltpu.VMEM((128,), jnp.int32)] * NBUF
            + [pltpu.VMEM((128, EMBED_DIM), jnp.float32)] * NBUF
            + [pltpu.VMEM((4, 8, 128), jnp.float32)] * NBUF
            + [pltpu.SemaphoreType.DMA] * NBUF
        ),
        compiler_params=pltpu.CompilerParams(
            use_tc_tiling_on_sc=False, needs_layout_passes=False
        ),
    )(idx_flat, t2)
    return out5.transpose(2, 4, 0, 1, 3).reshape(B_DIM, S_DIM, EMBED_DIM)
